# baseline (speedup 1.0000x reference)
import jax
import jax.numpy as jnp
from jax import lax
from jax.experimental import pallas as pl
from jax.experimental.pallas import tpu as pltpu
from jax.experimental.pallas import tpu_sc as plsc

EMBED_DIM = 64
LANES = 16
NUM_CORES = 2
NUM_SUBCORES = 16
NUM_WORKERS = NUM_CORES * NUM_SUBCORES
SLICES = EMBED_DIM // LANES
VOCAB = 53
GCHUNK = 4
GFETCH = 8


def _make_body(B, G):
    bpw = B // NUM_WORKERS
    lanes_pw = bpw // LANES

    def _body(idx_hbm, x_hbm, tab_hbm, w_hbm, b_hbm, out_hbm,
              tab_v, w_v, b_v, idx_v, x_v,
              rows_a, out_sem_a, rows_b, out_sem_b):
        wid = lax.axis_index("s") * NUM_CORES + lax.axis_index("c")
        b0 = pl.multiple_of(wid * bpw, 128)

        pltpu.sync_copy(w_hbm, w_v)
        pltpu.sync_copy(b_hbm, b_v)
        pltpu.sync_copy(tab_hbm, tab_v)
        pltpu.sync_copy(idx_hbm.at[:, pl.ds(b0, bpw)], idx_v)
        pltpu.sync_copy(x_hbm.at[:, pl.ds(b0, bpw)], x_v)
        w_regs = [w_v[pl.ds(c * LANES, LANES)] for c in range(SLICES)]
        b_regs = [b_v[pl.ds(c * LANES, LANES)] for c in range(SLICES)]

        def fold_row(v, _):
            for c in range(SLICES):
                sl = pl.ds(c * LANES, LANES)
                tab_v[v, sl] = tab_v[v, sl] + b_regs[c]
            return _

        lax.fori_loop(0, VOCAB, fold_row, None)

        lane_consts = [jnp.full((LANES,), l, jnp.int32) for l in range(LANES)]

        def store_wait(g0, rows_v, sem):
            pltpu.make_async_copy(
                rows_v, out_hbm.at[pl.ds(g0, GCHUNK), :, pl.ds(b0, bpw)],
                sem).wait()

        def chunk(g0, rows_v, sem):
            @pl.when(g0 >= 2 * GCHUNK)
            def _drain():
                store_wait(g0 - 2 * GCHUNK, rows_v, sem)

            @plsc.parallel_loop(0, GCHUNK, step=1)
            def g_body(gg):
                g = g0 + gg
                for bg in range(lanes_pw):
                    bsl = pl.ds(bg * LANES, LANES)
                    iv = idx_v[g, bsl]
                    xs = x_v[g, bsl]
                    for d in range(EMBED_DIM):
                        wd = jnp.take_along_axis(
                            w_regs[d // LANES], lane_consts[d % LANES],
                            axis=0)
                        tr = plsc.load_gather(
                            tab_v, [iv, jnp.full((LANES,), d, jnp.int32)])
                        rows_v[gg, d, bsl] = tr + xs * wd

            pltpu.async_copy(
                rows_v, out_hbm.at[pl.ds(g0, GCHUNK), :, pl.ds(b0, bpw)],
                sem)

        def pair_body(gp, _):
            chunk(gp * 2 * GCHUNK, rows_a, out_sem_a)
            chunk(gp * 2 * GCHUNK + GCHUNK, rows_b, out_sem_b)
            return _

        lax.fori_loop(0, G // (2 * GCHUNK), pair_body, None)
        store_wait(G - 2 * GCHUNK, rows_a, out_sem_a)
        store_wait(G - GCHUNK, rows_b, out_sem_b)

    return _body


def kernel(discrete_expression, normalized_expr, bin_table, W, b):
    B, G = discrete_expression.shape
    idx_t = discrete_expression.astype(jnp.int32).T
    x_t = normalized_expr.T
    w = W[:, 0]

    mesh = plsc.VectorSubcoreMesh(core_axis_name="c", subcore_axis_name="s")
    run = pl.kernel(
        _make_body(B, G),
        out_type=jax.ShapeDtypeStruct((G, EMBED_DIM, B), jnp.float32),
        mesh=mesh,
        compiler_params=pltpu.CompilerParams(needs_layout_passes=False),
        scratch_types=[
            pltpu.VMEM((VOCAB, EMBED_DIM), jnp.float32),
            pltpu.VMEM((EMBED_DIM,), jnp.float32),
            pltpu.VMEM((EMBED_DIM,), jnp.float32),
            pltpu.VMEM((G, B // NUM_WORKERS), jnp.int32),
            pltpu.VMEM((G, B // NUM_WORKERS), jnp.float32),
            pltpu.VMEM((GCHUNK, EMBED_DIM, B // NUM_WORKERS),
                       jnp.float32),
            pltpu.SemaphoreType.DMA,
            pltpu.VMEM((GCHUNK, EMBED_DIM, B // NUM_WORKERS),
                       jnp.float32),
            pltpu.SemaphoreType.DMA,
        ],
    )
    out = run(idx_t, x_t, bin_table, w, b)
    return out.transpose(2, 0, 1)


# speedup vs baseline: 1.7481x; 1.7481x over previous
import jax
import jax.numpy as jnp
from jax import lax
from jax.experimental import pallas as pl
from jax.experimental.pallas import tpu as pltpu
from jax.experimental.pallas import tpu_sc as plsc

EMBED_DIM = 64
LANES = 16
NUM_CORES = 2
NUM_SUBCORES = 16
NUM_WORKERS = NUM_CORES * NUM_SUBCORES
SLICES = EMBED_DIM // LANES
VOCAB = 53
GCHUNK = 4
GFETCH = 8


def _make_body(B, G):
    bpw = B // NUM_WORKERS
    lanes_pw = bpw // LANES

    def _body(idx_hbm, x_hbm, tab_hbm, w_hbm, b_hbm, out_hbm,
              tab_v, w_v, b_v, idx_v, x_v,
              rows_a, out_sem_a, rows_b, out_sem_b):
        wid = lax.axis_index("s") * NUM_CORES + lax.axis_index("c")
        b0 = pl.multiple_of(wid * bpw, 128)

        pltpu.sync_copy(w_hbm, w_v)
        pltpu.sync_copy(b_hbm, b_v)
        pltpu.sync_copy(tab_hbm, tab_v)
        pltpu.sync_copy(idx_hbm.at[:, pl.ds(b0, bpw)], idx_v)
        pltpu.sync_copy(x_hbm.at[:, pl.ds(b0, bpw)], x_v)
        w_regs = [w_v[pl.ds(c * LANES, LANES)] for c in range(SLICES)]
        b_regs = [b_v[pl.ds(c * LANES, LANES)] for c in range(SLICES)]

        def fold_row(v, _):
            for c in range(SLICES):
                sl = pl.ds(c * LANES, LANES)
                tab_v[v, sl] = tab_v[v, sl] + b_regs[c]
            return _

        lax.fori_loop(0, VOCAB, fold_row, None)

        lane_consts = [jnp.full((LANES,), l, jnp.int32) for l in range(LANES)]

        def store_wait(g0, rows_v, sem):
            pltpu.make_async_copy(
                rows_v, out_hbm.at[pl.ds(g0, GCHUNK), :, pl.ds(b0, bpw)],
                sem).wait()

        def chunk(g0, rows_v, sem):
            @pl.when(g0 >= 2 * GCHUNK)
            def _drain():
                store_wait(g0 - 2 * GCHUNK, rows_v, sem)

            @plsc.parallel_loop(0, GCHUNK * lanes_pw, step=1)
            def unit_body(u):
                gg = lax.shift_right_logical(u, 3)
                bg = lax.bitwise_and(u, lanes_pw - 1)
                bsl = pl.ds(bg * LANES, LANES)
                iv = idx_v[g0 + gg, bsl]
                xs = x_v[g0 + gg, bsl]
                for d in range(EMBED_DIM):
                    wd = jnp.take_along_axis(
                        w_regs[d // LANES], lane_consts[d % LANES], axis=0)
                    tr = plsc.load_gather(
                        tab_v, [iv, jnp.full((LANES,), d, jnp.int32)])
                    rows_v[gg, d, bsl] = tr + xs * wd

            pltpu.async_copy(
                rows_v, out_hbm.at[pl.ds(g0, GCHUNK), :, pl.ds(b0, bpw)],
                sem)

        def pair_body(gp, _):
            chunk(gp * 2 * GCHUNK, rows_a, out_sem_a)
            chunk(gp * 2 * GCHUNK + GCHUNK, rows_b, out_sem_b)
            return _

        lax.fori_loop(0, G // (2 * GCHUNK), pair_body, None)
        store_wait(G - 2 * GCHUNK, rows_a, out_sem_a)
        store_wait(G - GCHUNK, rows_b, out_sem_b)

    return _body


def kernel(discrete_expression, normalized_expr, bin_table, W, b):
    B, G = discrete_expression.shape
    idx_t = discrete_expression.astype(jnp.int32).T
    x_t = normalized_expr.T
    w = W[:, 0]

    mesh = plsc.VectorSubcoreMesh(core_axis_name="c", subcore_axis_name="s")
    run = pl.kernel(
        _make_body(B, G),
        out_type=jax.ShapeDtypeStruct((G, EMBED_DIM, B), jnp.float32),
        mesh=mesh,
        compiler_params=pltpu.CompilerParams(needs_layout_passes=False),
        scratch_types=[
            pltpu.VMEM((VOCAB, EMBED_DIM), jnp.float32),
            pltpu.VMEM((EMBED_DIM,), jnp.float32),
            pltpu.VMEM((EMBED_DIM,), jnp.float32),
            pltpu.VMEM((G, B // NUM_WORKERS), jnp.int32),
            pltpu.VMEM((G, B // NUM_WORKERS), jnp.float32),
            pltpu.VMEM((GCHUNK, EMBED_DIM, B // NUM_WORKERS),
                       jnp.float32),
            pltpu.SemaphoreType.DMA,
            pltpu.VMEM((GCHUNK, EMBED_DIM, B // NUM_WORKERS),
                       jnp.float32),
            pltpu.SemaphoreType.DMA,
        ],
    )
    out = run(idx_t, x_t, bin_table, w, b)
    return out.transpose(2, 0, 1)


# speedup vs baseline: 3.9868x; 2.2807x over previous
import jax
import jax.numpy as jnp
from jax import lax
from jax.experimental import pallas as pl
from jax.experimental.pallas import tpu as pltpu
from jax.experimental.pallas import tpu_sc as plsc

EMBED_DIM = 64
LANES = 16
NUM_CORES = 2
NUM_SUBCORES = 16
NUM_WORKERS = NUM_CORES * NUM_SUBCORES
CHUNK = 256
SLICES = EMBED_DIM // LANES
VOCAB = 53


def _body(idx_hbm, x_hbm, tab_hbm, w_hbm, b_hbm, out_hbm,
          tab_v, w_v, b_v,
          idx_a, x_a, rows_a, in_sem_a, out_sem_a,
          idx_b, x_b, rows_b, in_sem_b, out_sem_b):
    wid = lax.axis_index("s") * NUM_CORES + lax.axis_index("c")
    rows_per_worker = out_hbm.shape[0] // NUM_WORKERS
    n_chunks = rows_per_worker // CHUNK
    worker_base = wid * rows_per_worker

    pltpu.sync_copy(w_hbm, w_v)
    pltpu.sync_copy(b_hbm, b_v)
    pltpu.sync_copy(tab_hbm, tab_v)
    w_regs = [w_v[pl.ds(c * LANES, LANES)] for c in range(SLICES)]
    b_regs = [b_v[pl.ds(c * LANES, LANES)] for c in range(SLICES)]

    def fold_row(v, _):
        for c in range(SLICES):
            sl = pl.ds(c * LANES, LANES)
            tab_v[v, sl] = tab_v[v, sl] + b_regs[c]
        return _

    lax.fori_loop(0, VOCAB, fold_row, None)

    col_regs = [c * LANES + lax.iota(jnp.int32, LANES) for c in range(SLICES)]

    def fetch(ci, idx_v, x_v, sem):
        base = worker_base + ci * CHUNK
        pltpu.async_copy(idx_hbm.at[pl.ds(base, CHUNK)], idx_v, sem)
        pltpu.async_copy(x_hbm.at[pl.ds(base, CHUNK)], x_v, sem)

    def fetch_wait(ci, idx_v, x_v, sem):
        base = worker_base + ci * CHUNK
        pltpu.make_async_copy(idx_hbm.at[pl.ds(base, CHUNK)], idx_v,
                              sem).wait()
        pltpu.make_async_copy(x_hbm.at[pl.ds(base, CHUNK)], x_v, sem).wait()

    def store_wait(ci, rows_v, sem):
        base = worker_base + ci * CHUNK
        pltpu.make_async_copy(rows_v, out_hbm.at[pl.ds(base, CHUNK)],
                              sem).wait()

    def process(ci, idx_v, x_v, rows_v, in_sem, out_sem):
        fetch_wait(ci, idx_v, x_v, in_sem)

        @pl.when(ci >= 2)
        def _drain():
            store_wait(ci - 2, rows_v, out_sem)

        @plsc.parallel_loop(0, CHUNK, step=1, unroll=8)
        def row_body(r):
            lane_r = jnp.broadcast_to(r, (LANES,))
            iv = plsc.load_gather(idx_v, [lane_r])
            xs = plsc.load_gather(x_v, [lane_r])
            for c in range(SLICES):
                tr = plsc.load_gather(tab_v, [iv, col_regs[c]])
                rows_v[r, pl.ds(c * LANES, LANES)] = tr + xs * w_regs[c]

        base = worker_base + ci * CHUNK
        pltpu.async_copy(rows_v, out_hbm.at[pl.ds(base, CHUNK)], out_sem)

        @pl.when(ci + 2 < n_chunks)
        def _prefetch():
            fetch(ci + 2, idx_v, x_v, in_sem)

    fetch(0, idx_a, x_a, in_sem_a)
    fetch(1, idx_b, x_b, in_sem_b)

    def pair_body(cp, _):
        process(cp * 2, idx_a, x_a, rows_a, in_sem_a, out_sem_a)
        process(cp * 2 + 1, idx_b, x_b, rows_b, in_sem_b, out_sem_b)
        return _

    lax.fori_loop(0, n_chunks // 2, pair_body, None)
    store_wait(n_chunks - 2, rows_a, out_sem_a)
    store_wait(n_chunks - 1, rows_b, out_sem_b)


def kernel(discrete_expression, normalized_expr, bin_table, W, b):
    B, G = discrete_expression.shape
    N = B * G
    idx = discrete_expression.astype(jnp.int32).reshape(N)
    x = normalized_expr.reshape(N)
    w = W[:, 0]

    mesh = plsc.VectorSubcoreMesh(core_axis_name="c", subcore_axis_name="s")
    run = pl.kernel(
        _body,
        out_type=jax.ShapeDtypeStruct((N, EMBED_DIM), jnp.float32),
        mesh=mesh,
        compiler_params=pltpu.CompilerParams(needs_layout_passes=False),
        scratch_types=[
            pltpu.VMEM((VOCAB, EMBED_DIM), jnp.float32),
            pltpu.VMEM((EMBED_DIM,), jnp.float32),
            pltpu.VMEM((EMBED_DIM,), jnp.float32),
            pltpu.VMEM((CHUNK,), jnp.int32),
            pltpu.VMEM((CHUNK,), jnp.float32),
            pltpu.VMEM((CHUNK, EMBED_DIM), jnp.float32),
            pltpu.SemaphoreType.DMA,
            pltpu.SemaphoreType.DMA,
            pltpu.VMEM((CHUNK,), jnp.int32),
            pltpu.VMEM((CHUNK,), jnp.float32),
            pltpu.VMEM((CHUNK, EMBED_DIM), jnp.float32),
            pltpu.SemaphoreType.DMA,
            pltpu.SemaphoreType.DMA,
        ],
    )
    out = run(idx, x, bin_table, w, b)
    return out.reshape(B, G, EMBED_DIM)
